# Initial kernel scaffold; baseline (speedup 1.0000x reference)
import jax
import jax.numpy as jnp
from jax.experimental import pallas as pl


def kernel(input):
    raise NotImplementedError("write your pallas kernel here")


# speedup vs baseline: 1.4470x; 1.4470x over previous
import jax
import jax.numpy as jnp
from jax import lax
from jax.experimental import pallas as pl
from jax.experimental.pallas import tpu as pltpu
from jax.experimental.pallas import tpu_sc as plsc

_NUM_CLASSES = 10
_MAX_STEPS = 10
_T = 8388608
_NUM_STEPS = int(min(_MAX_STEPS, _T // 2))

_NC, _NS, _L = 2, 16, 16
_NW = _NC * _NS
_PW = _T // _NW
_CH = 8192
_NCHUNK = _PW // _CH


def _body(x_hbm, b_hbm, cm_hbm, out_hbm, b_v, cm_v, xbuf, obuf):
    wid = lax.axis_index("s") * _NC + lax.axis_index("c")
    pltpu.sync_copy(b_hbm, b_v)
    pltpu.sync_copy(cm_hbm, cm_v)
    bvec = b_v[...]
    cmvec = cm_v[...]
    bs = [jnp.full((_L,), bvec[j], jnp.float32) for j in range(_NUM_STEPS - 1)]
    cms = [jnp.full((_L,), cmvec[j], jnp.int32) for j in range(_NUM_STEPS)]
    base = wid * _PW

    def chunk_body(ci, _):
        off = base + ci * _CH
        pltpu.sync_copy(x_hbm.at[pl.ds(off, _CH)], xbuf)

        def vec_body(vi, _):
            x = xbuf[pl.ds(vi * _L, _L)]
            acc = cms[0]
            for j in range(_NUM_STEPS - 1):
                acc = jnp.where(x > bs[j], cms[j + 1], acc)
            obuf[pl.ds(vi * _L, _L)] = acc
            return 0

        lax.fori_loop(0, _CH // _L, vec_body, 0, unroll=4)
        pltpu.sync_copy(obuf, out_hbm.at[pl.ds(off, _CH)])
        return 0

    lax.fori_loop(0, _NCHUNK, chunk_body, 0)


def kernel(input):
    key = jax.random.key(42)
    k1, k2 = jax.random.split(key)
    bidx = jax.random.randint(k1, (_NUM_STEPS - 1,), 0, _T)
    boundaries = jnp.sort(input[bidx])
    class_mapping = jax.random.randint(k2, (_NUM_STEPS,), 0, _NUM_CLASSES)
    b16 = jnp.zeros((_L,), jnp.float32).at[: _NUM_STEPS - 1].set(boundaries)
    cm16 = jnp.zeros((_L,), jnp.int32).at[: _NUM_STEPS].set(class_mapping)

    mesh = plsc.VectorSubcoreMesh(core_axis_name="c", subcore_axis_name="s")
    run = pl.kernel(
        _body,
        out_type=jax.ShapeDtypeStruct((_T,), jnp.int32),
        mesh=mesh,
        scratch_types=[
            pltpu.VMEM((_L,), jnp.float32),
            pltpu.VMEM((_L,), jnp.int32),
            pltpu.VMEM((_CH,), jnp.float32),
            pltpu.VMEM((_CH,), jnp.int32),
        ],
    )
    return run(input, b16, cm16)


# speedup vs baseline: 1.7700x; 1.2232x over previous
import jax
import jax.numpy as jnp
from jax import lax
from jax.experimental import pallas as pl
from jax.experimental.pallas import tpu as pltpu
from jax.experimental.pallas import tpu_sc as plsc

_NUM_CLASSES = 10
_MAX_STEPS = 10
_T = 8388608
_NUM_STEPS = int(min(_MAX_STEPS, _T // 2))

_NC, _NS, _L = 2, 16, 16
_NW = _NC * _NS
_PW = _T // _NW
_CH = 16384
_NCHUNK = _PW // _CH
_NPAIR = _NCHUNK // 2


def _body(x_hbm, b_hbm, cm_hbm, out_hbm,
          b_v, cm_v, xb0, xb1, ob0, ob1, si0, si1, so0, so1):
    xbufs, obufs = (xb0, xb1), (ob0, ob1)
    sin, sout = (si0, si1), (so0, so1)
    wid = lax.axis_index("s") * _NC + lax.axis_index("c")
    base = wid * _PW

    for b in range(2):
        pltpu.async_copy(x_hbm.at[pl.ds(base + b * _CH, _CH)], xbufs[b],
                         sin[b])

    pltpu.sync_copy(b_hbm, b_v)
    pltpu.sync_copy(cm_hbm, cm_v)
    bvec = b_v[...]
    cmvec = cm_v[...]
    bs = [jnp.full((_L,), bvec[j], jnp.float32) for j in range(_NUM_STEPS - 1)]
    cms = [jnp.full((_L,), cmvec[j], jnp.int32) for j in range(_NUM_STEPS)]

    def pair_body(p, _):
        for b in range(2):
            off = base + (2 * p + b) * _CH
            pltpu.make_async_copy(
                x_hbm.at[pl.ds(off, _CH)], xbufs[b], sin[b]).wait()

            @pl.when(p > 0)
            def _wait_out():
                pltpu.make_async_copy(
                    obufs[b], out_hbm.at[pl.ds(off - 2 * _CH, _CH)],
                    sout[b]).wait()

            def vec_body(vi, _, b=b):
                x = xbufs[b][pl.ds(vi * _L, _L)]
                acc = cms[0]
                for j in range(_NUM_STEPS - 1):
                    acc = jnp.where(x > bs[j], cms[j + 1], acc)
                obufs[b][pl.ds(vi * _L, _L)] = acc
                return 0

            lax.fori_loop(0, _CH // _L, vec_body, 0, unroll=8)
            pltpu.async_copy(obufs[b], out_hbm.at[pl.ds(off, _CH)], sout[b])

            @pl.when(p < _NPAIR - 1)
            def _issue_next():
                pltpu.async_copy(
                    x_hbm.at[pl.ds(off + 2 * _CH, _CH)], xbufs[b], sin[b])
        return 0

    lax.fori_loop(0, _NPAIR, pair_body, 0)
    for b in range(2):
        pltpu.make_async_copy(
            obufs[b],
            out_hbm.at[pl.ds(base + (_NCHUNK - 2 + b) * _CH, _CH)],
            sout[b]).wait()


def kernel(input):
    key = jax.random.key(42)
    k1, k2 = jax.random.split(key)
    bidx = jax.random.randint(k1, (_NUM_STEPS - 1,), 0, _T)
    boundaries = jnp.sort(input[bidx])
    class_mapping = jax.random.randint(k2, (_NUM_STEPS,), 0, _NUM_CLASSES)
    b16 = jnp.zeros((_L,), jnp.float32).at[: _NUM_STEPS - 1].set(boundaries)
    cm16 = jnp.zeros((_L,), jnp.int32).at[: _NUM_STEPS].set(class_mapping)

    mesh = plsc.VectorSubcoreMesh(core_axis_name="c", subcore_axis_name="s")
    run = pl.kernel(
        _body,
        out_type=jax.ShapeDtypeStruct((_T,), jnp.int32),
        mesh=mesh,
        scratch_types=[
            pltpu.VMEM((_L,), jnp.float32),
            pltpu.VMEM((_L,), jnp.int32),
            pltpu.VMEM((_CH,), jnp.float32),
            pltpu.VMEM((_CH,), jnp.float32),
            pltpu.VMEM((_CH,), jnp.int32),
            pltpu.VMEM((_CH,), jnp.int32),
            pltpu.SemaphoreType.DMA,
            pltpu.SemaphoreType.DMA,
            pltpu.SemaphoreType.DMA,
            pltpu.SemaphoreType.DMA,
        ],
    )
    return run(input, b16, cm16)


# speedup vs baseline: 3.3553x; 1.8957x over previous
import jax
import jax.numpy as jnp
from jax import lax
from jax.experimental import pallas as pl
from jax.experimental.pallas import tpu as pltpu
from jax.experimental.pallas import tpu_sc as plsc

_NUM_CLASSES = 10
_MAX_STEPS = 10
_T = 8388608
_NUM_STEPS = int(min(_MAX_STEPS, _T // 2))

_NC, _NS, _L = 2, 16, 16
_NW = _NC * _NS
_PW = _T // _NW
_CH = 16384
_NCHUNK = _PW // _CH
_NPAIR = _NCHUNK // 2


def _body(x_hbm, b_hbm, cm_hbm, out_hbm,
          b_v, cm_v, xb0, xb1, ob0, ob1, si0, si1, so0, so1):
    xbufs, obufs = (xb0, xb1), (ob0, ob1)
    sin, sout = (si0, si1), (so0, so1)
    wid = lax.axis_index("s") * _NC + lax.axis_index("c")
    base = wid * _PW

    for b in range(2):
        pltpu.async_copy(x_hbm.at[pl.ds(base + b * _CH, _CH)], xbufs[b],
                         sin[b])

    pltpu.sync_copy(b_hbm, b_v)
    pltpu.sync_copy(cm_hbm, cm_v)
    bvec = b_v[...]
    cmvec = cm_v[...]
    bs = [jnp.full((_L,), bvec[j], jnp.float32) for j in range(_NUM_STEPS - 1)]
    cms = [jnp.full((_L,), cmvec[j], jnp.int32) for j in range(_NUM_STEPS)]

    def pair_body(p, _):
        for b in range(2):
            off = base + (2 * p + b) * _CH
            pltpu.make_async_copy(
                x_hbm.at[pl.ds(off, _CH)], xbufs[b], sin[b]).wait()

            @pl.when(p > 0)
            def _wait_out():
                pltpu.make_async_copy(
                    obufs[b], out_hbm.at[pl.ds(off - 2 * _CH, _CH)],
                    sout[b]).wait()

            @plsc.parallel_loop(0, _CH, step=_L, unroll=8)
            def _vecs(i, b=b):
                x = xbufs[b][pl.ds(i, _L)]
                acc = cms[0]
                for j in range(_NUM_STEPS - 1):
                    acc = jnp.where(x > bs[j], cms[j + 1], acc)
                obufs[b][pl.ds(i, _L)] = acc
            pltpu.async_copy(obufs[b], out_hbm.at[pl.ds(off, _CH)], sout[b])

            @pl.when(p < _NPAIR - 1)
            def _issue_next():
                pltpu.async_copy(
                    x_hbm.at[pl.ds(off + 2 * _CH, _CH)], xbufs[b], sin[b])
        return 0

    lax.fori_loop(0, _NPAIR, pair_body, 0)
    for b in range(2):
        pltpu.make_async_copy(
            obufs[b],
            out_hbm.at[pl.ds(base + (_NCHUNK - 2 + b) * _CH, _CH)],
            sout[b]).wait()


def kernel(input):
    key = jax.random.key(42)
    k1, k2 = jax.random.split(key)
    bidx = jax.random.randint(k1, (_NUM_STEPS - 1,), 0, _T)
    boundaries = jnp.sort(input[bidx])
    class_mapping = jax.random.randint(k2, (_NUM_STEPS,), 0, _NUM_CLASSES)
    b16 = jnp.zeros((_L,), jnp.float32).at[: _NUM_STEPS - 1].set(boundaries)
    cm16 = jnp.zeros((_L,), jnp.int32).at[: _NUM_STEPS].set(class_mapping)

    mesh = plsc.VectorSubcoreMesh(core_axis_name="c", subcore_axis_name="s")
    run = pl.kernel(
        _body,
        out_type=jax.ShapeDtypeStruct((_T,), jnp.int32),
        mesh=mesh,
        scratch_types=[
            pltpu.VMEM((_L,), jnp.float32),
            pltpu.VMEM((_L,), jnp.int32),
            pltpu.VMEM((_CH,), jnp.float32),
            pltpu.VMEM((_CH,), jnp.float32),
            pltpu.VMEM((_CH,), jnp.int32),
            pltpu.VMEM((_CH,), jnp.int32),
            pltpu.SemaphoreType.DMA,
            pltpu.SemaphoreType.DMA,
            pltpu.SemaphoreType.DMA,
            pltpu.SemaphoreType.DMA,
        ],
    )
    return run(input, b16, cm16)
